# baseline (speedup 1.0000x reference)
import functools

import jax
import jax.numpy as jnp
from jax import lax
from jax.experimental import pallas as pl
from jax.experimental.pallas import tpu as pltpu
from jax.experimental.pallas import tpu_sc as plsc

B, C, H, W, K = 32, 64, 128, 128, 128
HW = H * W
NC, NS, L = 2, 16, 16
NW = NC * NS
EPW = K * C
KB = K // L
RCAP = K + L
GCH = 128
WEIGHT_ANGLE = 10.0

_mesh = plsc.VectorSubcoreMesh(core_axis_name="c", subcore_axis_name="s")


@functools.partial(
    pl.kernel,
    mesh=_mesh,
    out_type=(
        jax.ShapeDtypeStruct((NW, L), jnp.float32),
        jax.ShapeDtypeStruct((NW, L), jnp.float32),
    ),
    scratch_types=[
        pltpu.VMEM((K,), jnp.int32),
        pltpu.VMEM((K,), jnp.int32),
        pltpu.VMEM((RCAP * C,), jnp.int32),
        pltpu.VMEM((RCAP * C,), jnp.float32),
        pltpu.VMEM((EPW,), jnp.float32),
        pltpu.VMEM((RCAP * L,), jnp.int32),
        pltpu.VMEM((RCAP * L,), jnp.float32),
        pltpu.VMEM((L,), jnp.float32),
        pltpu.VMEM((L,), jnp.float32),
        pltpu.SemaphoreType.DMA,
        pltpu.SemaphoreType.DMA,
    ],
)
def _sc_partials(out_hbm, ind_hbm, mask_hbm, tgt_hbm,
                 psum_hbm, pcnt_hbm,
                 ind_v, mask_v, idx_v, pred_v, tgt_v, toff_v, coef_v,
                 psum_v, pcnt_v, sem_t, sem_g):
    wid = lax.axis_index("s") * NC + lax.axis_index("c")

    cp_t = pltpu.async_copy(tgt_hbm.at[pl.ds(wid * EPW, EPW)], tgt_v, sem_t)
    pltpu.sync_copy(ind_hbm.at[pl.ds(wid * K, K)], ind_v)
    pltpu.sync_copy(mask_hbm.at[pl.ds(wid * K, K)], mask_v)

    lanes = lax.iota(jnp.int32, L)
    base = wid * (C * HW)
    wvec = jnp.where(lanes % 2 == 0,
                     jnp.full((L,), 1.0, jnp.float32),
                     jnp.full((L,), WEIGHT_ANGLE, jnp.float32))
    lhw = [(lanes + cb * L) * HW for cb in range(C // L)]
    zf = jnp.zeros((L,), jnp.float32)
    zi = jnp.zeros((L,), jnp.int32)

    NCH = RCAP * C // GCH
    NB = RCAP // L

    def zmeta(j, x):
        toff_v[pl.ds(j * L, L)] = zi
        coef_v[pl.ds(j * L, L)] = zf
        return x

    lax.fori_loop(0, RCAP, zmeta, 0)

    def zpred(j, x):
        pred_v[pl.ds(j * L, L)] = zf
        return x

    lax.fori_loop(0, RCAP * C // L, zpred, 0)

    cur = jnp.int32(0)
    cnt = zf
    for kb in range(KB):
        vk = ind_v[pl.ds(kb * L, L)] + base
        vm = mask_v[pl.ds(kb * L, L)]
        mf = vm.astype(jnp.float32)
        coefv = mf * wvec
        cnt = cnt + mf
        for u in range(L):
            sk = jnp.full((L,), vk[u], jnp.int32)
            ebase = cur * C
            for cb in range(C // L):
                idx_v[pl.ds(ebase + cb * L, L)] = sk + lhw[cb]
            rbase = cur * L
            toff_v[pl.ds(rbase, L)] = jnp.full((L,), (kb * L + u) * C, jnp.int32)
            coef_v[pl.ds(rbase, L)] = jnp.full((L,), coefv[u], jnp.float32)
            cur = cur + jnp.where(vm[u] != 0, 1, 0).astype(jnp.int32)

    for r in range(L):
        ebase = (cur + r) * C
        for cb in range(C // L):
            idx_v[pl.ds(ebase + cb * L, L)] = zi
        rbase = (cur + r) * L
        toff_v[pl.ds(rbase, L)] = zi
        coef_v[pl.ds(rbase, L)] = zf

    nb = lax.div(cur + (L - 1), jnp.int32(L))
    nch = nb * (L * C // GCH)

    for j in range(NCH):
        @pl.when(j < nch)
        def _():
            pltpu.async_copy(out_hbm.at[idx_v.at[pl.ds(j * GCH, GCH)]],
                             pred_v.at[pl.ds(j * GCH, GCH)], sem_g)

    for j in range(NCH):
        @pl.when(j < nch)
        def _():
            pltpu.make_async_copy(out_hbm.at[pl.ds(0, GCH)],
                                  pred_v.at[pl.ds(0, GCH)], sem_g).wait()

    cp_t.wait()

    acc = zf
    for jb in range(NB):
        contrib = zf
        for u in range(L):
            rbase = jb * (L * L) + u * L
            cf = coef_v[pl.ds(rbase, L)]
            t0 = jnp.clip(toff_v[pl.ds(rbase, L)][0], 0, EPW - C)
            ebase = jb * (L * C) + u * C
            for cb in range(C // L):
                pr = pred_v[pl.ds(ebase + cb * L, L)]
                tg = tgt_v[pl.ds(t0 + cb * L, L)]
                contrib = contrib + cf * jnp.abs(pr - tg)
        acc = acc + contrib

    psum_v[...] = acc
    pcnt_v[...] = cnt
    pltpu.sync_copy(psum_v, psum_hbm.at[wid])
    pltpu.sync_copy(pcnt_v, pcnt_hbm.at[wid])


def _finish_body(ps_ref, pc_ref, o_ref):
    total = jnp.sum(ps_ref[...])
    count = jnp.sum(pc_ref[...])
    o_ref[...] = jnp.broadcast_to(total / (count * float(C) + 1e-4), (1, 1))


_finish = pl.pallas_call(
    _finish_body,
    out_shape=jax.ShapeDtypeStruct((1, 1), jnp.float32),
)


def kernel(output, mask, ind, target, freq_mask):
    del freq_mask
    psum, pcnt = _sc_partials(
        output.reshape(-1),
        ind.reshape(-1).astype(jnp.int32),
        mask.reshape(-1).astype(jnp.int32),
        target.reshape(-1),
    )
    return _finish(psum, pcnt)[0, 0]


# speedup vs baseline: 3.2060x; 3.2060x over previous
import functools

import jax
import jax.numpy as jnp
from jax import lax
from jax.experimental import pallas as pl
from jax.experimental.pallas import tpu as pltpu
from jax.experimental.pallas import tpu_sc as plsc

B, C, H, W, K = 32, 64, 128, 128, 128
HW = H * W
NC, NS, L = 2, 16, 16
NW = NC * NS
EPW = K * C
KB = K // L
BLK = L * C
GCH = 128
WEIGHT_ANGLE = 10.0

_mesh = plsc.VectorSubcoreMesh(core_axis_name="c", subcore_axis_name="s")


@functools.partial(
    pl.kernel,
    mesh=_mesh,
    out_type=(
        jax.ShapeDtypeStruct((NW, L), jnp.float32),
        jax.ShapeDtypeStruct((NW, L), jnp.float32),
    ),
    scratch_types=[
        pltpu.VMEM((K,), jnp.int32),
        pltpu.VMEM((K,), jnp.int32),
        pltpu.VMEM((EPW,), jnp.int32),
        pltpu.VMEM((EPW,), jnp.float32),
        pltpu.VMEM((EPW,), jnp.float32),
        pltpu.VMEM((L,), jnp.float32),
        pltpu.VMEM((L,), jnp.float32),
        pltpu.SemaphoreType.DMA,
        pltpu.SemaphoreType.DMA,
        pltpu.SemaphoreType.DMA,
    ],
)
def _sc_partials(out_hbm, ind_hbm, mask_hbm, tgt_hbm,
                 psum_hbm, pcnt_hbm,
                 ind_v, mask_v, idx_v, pred_v, tgt_v,
                 psum_v, pcnt_v, sem_t, sem_a, sem_b):
    wid = lax.axis_index("s") * NC + lax.axis_index("c")

    cp_t = pltpu.async_copy(tgt_hbm.at[pl.ds(wid * EPW, EPW)], tgt_v, sem_t)
    pltpu.sync_copy(ind_hbm.at[pl.ds(wid * K, K)], ind_v)
    pltpu.sync_copy(mask_hbm.at[pl.ds(wid * K, K)], mask_v)

    lanes = lax.iota(jnp.int32, L)
    base = wid * (C * HW)
    wvec = jnp.where(lanes % 2 == 0,
                     jnp.full((L,), 1.0, jnp.float32),
                     jnp.full((L,), WEIGHT_ANGLE, jnp.float32))
    lhw = [(lanes + cb * L) * HW for cb in range(C // L)]
    sems = (sem_a, sem_b)

    def build(kb):
        vk = ind_v[pl.ds(kb * L, L)] + base
        for u in range(L):
            sk = vk[u]
            for cb in range(C // L):
                e = kb * BLK + u * C + cb * L
                idx_v[pl.ds(e, L)] = lhw[cb] + sk
        return pltpu.async_copy(out_hbm.at[idx_v.at[pl.ds(kb * BLK, BLK)]],
                                pred_v.at[pl.ds(kb * BLK, BLK)], sems[kb % 2])

    def compute(kb, acc, cnt):
        mf = mask_v[pl.ds(kb * L, L)].astype(jnp.float32)
        coefv = mf * wvec
        for u in range(L):
            cf = jnp.full((L,), coefv[u], jnp.float32)
            for cb in range(C // L):
                e = kb * BLK + u * C + cb * L
                d = pred_v[pl.ds(e, L)] - tgt_v[pl.ds(e, L)]
                acc = acc + cf * jnp.abs(d)
        return acc, cnt + mf

    acc = jnp.zeros((L,), jnp.float32)
    cnt = jnp.zeros((L,), jnp.float32)
    cps = [build(0)]
    cp_t.wait()
    for kb in range(1, KB):
        cps.append(build(kb))
        cps[kb - 1].wait()
        acc, cnt = compute(kb - 1, acc, cnt)
    cps[KB - 1].wait()
    acc, cnt = compute(KB - 1, acc, cnt)

    psum_v[...] = acc
    pcnt_v[...] = cnt
    pltpu.sync_copy(psum_v, psum_hbm.at[wid])
    pltpu.sync_copy(pcnt_v, pcnt_hbm.at[wid])


def _finish_body(ps_ref, pc_ref, o_ref):
    total = jnp.sum(ps_ref[...])
    count = jnp.sum(pc_ref[...])
    o_ref[...] = jnp.broadcast_to(total / (count * float(C) + 1e-4), (1, 1))


_finish = pl.pallas_call(
    _finish_body,
    out_shape=jax.ShapeDtypeStruct((1, 1), jnp.float32),
)


def kernel(output, mask, ind, target, freq_mask):
    del freq_mask
    psum, pcnt = _sc_partials(
        output.reshape(-1),
        ind.reshape(-1).astype(jnp.int32),
        mask.reshape(-1).astype(jnp.int32),
        target.reshape(-1),
    )
    return _finish(psum, pcnt)[0, 0]


# speedup vs baseline: 3.2495x; 1.0135x over previous
import functools

import jax
import jax.numpy as jnp
from jax import lax
from jax.experimental import pallas as pl
from jax.experimental.pallas import tpu as pltpu
from jax.experimental.pallas import tpu_sc as plsc

B, C, H, W, K = 32, 64, 128, 128, 128
HW = H * W
NC, NS, L = 2, 16, 16
NW = NC * NS
EPW = K * C
CB = 4
BLK = EPW // CB
WEIGHT_ANGLE = 10.0

_mesh = plsc.VectorSubcoreMesh(core_axis_name="c", subcore_axis_name="s")


@functools.partial(
    pl.kernel,
    mesh=_mesh,
    out_type=(
        jax.ShapeDtypeStruct((NW, L), jnp.float32),
        jax.ShapeDtypeStruct((NW, L), jnp.float32),
    ),
    scratch_types=[
        pltpu.VMEM((K,), jnp.int32),
        pltpu.VMEM((K,), jnp.int32),
        pltpu.VMEM((EPW,), jnp.int32),
        pltpu.VMEM((EPW,), jnp.float32),
        pltpu.VMEM((EPW,), jnp.float32),
        pltpu.VMEM((L,), jnp.float32),
        pltpu.VMEM((L,), jnp.float32),
        pltpu.SemaphoreType.DMA,
        pltpu.SemaphoreType.DMA,
        pltpu.SemaphoreType.DMA,
    ],
)
def _sc_partials(out_hbm, ind_hbm, mask_hbm, tgt_hbm,
                 psum_hbm, pcnt_hbm,
                 ind_v, mask_v, idx_v, pred_v, tgt_v,
                 psum_v, pcnt_v, sem_t, sem_a, sem_b):
    wid = lax.axis_index("s") * NC + lax.axis_index("c")

    cp_t = pltpu.async_copy(tgt_hbm.at[pl.ds(wid * EPW, EPW)], tgt_v, sem_t)
    pltpu.sync_copy(ind_hbm.at[pl.ds(wid * K, K)], ind_v)
    pltpu.sync_copy(mask_hbm.at[pl.ds(wid * K, K)], mask_v)

    lanes = lax.iota(jnp.int32, L)
    base = wid * (C * HW)
    wvec = jnp.where(lanes % 2 == 0,
                     jnp.full((L,), 1.0, jnp.float32),
                     jnp.full((L,), WEIGHT_ANGLE, jnp.float32))
    sems = (sem_a, sem_b)

    vks = [ind_v[pl.ds(g * L, L)] + base for g in range(K // L)]
    mfs = [mask_v[pl.ds(g * L, L)].astype(jnp.float32) for g in range(K // L)]
    coefs = [mf * wvec for mf in mfs]
    cnt = mfs[0]
    for mf in mfs[1:]:
        cnt = cnt + mf

    def build(j):
        for cl in range(BLK // K):
            c = j * (BLK // K) + cl
            for g in range(K // L):
                idx_v[pl.ds(c * K + g * L, L)] = vks[g] + c * HW
        return pltpu.async_copy(out_hbm.at[idx_v.at[pl.ds(j * BLK, BLK)]],
                                pred_v.at[pl.ds(j * BLK, BLK)], sems[j % 2])

    def compute(j, acc):
        for cl in range(BLK // K):
            c = j * (BLK // K) + cl
            for g in range(K // L):
                off = c * K + g * L
                d = pred_v[pl.ds(off, L)] - tgt_v[pl.ds(off, L)]
                acc = acc + coefs[g] * jnp.abs(d)
        return acc

    acc = jnp.zeros((L,), jnp.float32)
    cps = [build(0)]
    cp_t.wait()
    for j in range(1, CB):
        cps.append(build(j))
        cps[j - 1].wait()
        acc = compute(j - 1, acc)
    cps[CB - 1].wait()
    acc = compute(CB - 1, acc)

    psum_v[...] = acc
    pcnt_v[...] = cnt
    pltpu.sync_copy(psum_v, psum_hbm.at[wid])
    pltpu.sync_copy(pcnt_v, pcnt_hbm.at[wid])


def _finish_body(ps_ref, pc_ref, o_ref):
    total = jnp.sum(ps_ref[...])
    count = jnp.sum(pc_ref[...])
    o_ref[...] = jnp.broadcast_to(total / (count * float(C) + 1e-4), (1, 1))


_finish = pl.pallas_call(
    _finish_body,
    out_shape=jax.ShapeDtypeStruct((1, 1), jnp.float32),
)


def kernel(output, mask, ind, target, freq_mask):
    del freq_mask
    psum, pcnt = _sc_partials(
        output.reshape(-1),
        ind.reshape(-1).astype(jnp.int32),
        mask.reshape(-1).astype(jnp.int32),
        target.transpose(0, 2, 1).reshape(-1),
    )
    return _finish(psum, pcnt)[0, 0]


# speedup vs baseline: 3.3295x; 1.0246x over previous
import functools

import jax
import jax.numpy as jnp
from jax import lax
from jax.experimental import pallas as pl
from jax.experimental.pallas import tpu as pltpu
from jax.experimental.pallas import tpu_sc as plsc

B, C, H, W, K = 32, 64, 128, 128, 128
HW = H * W
NC, NS, L = 2, 16, 16
NW = NC * NS
EPW = K * C
CB = 8
BLK = EPW // CB
WEIGHT_ANGLE = 10.0

_mesh = plsc.VectorSubcoreMesh(core_axis_name="c", subcore_axis_name="s")


@functools.partial(
    pl.kernel,
    mesh=_mesh,
    out_type=(
        jax.ShapeDtypeStruct((NW, L), jnp.float32),
        jax.ShapeDtypeStruct((NW, L), jnp.float32),
    ),
    scratch_types=[
        pltpu.VMEM((K,), jnp.int32),
        pltpu.VMEM((K,), jnp.int32),
        pltpu.VMEM((EPW,), jnp.int32),
        pltpu.VMEM((EPW,), jnp.float32),
        pltpu.VMEM((EPW,), jnp.float32),
        pltpu.VMEM((L,), jnp.float32),
        pltpu.VMEM((L,), jnp.float32),
        pltpu.SemaphoreType.DMA,
        pltpu.SemaphoreType.DMA,
        pltpu.SemaphoreType.DMA,
        pltpu.SemaphoreType.DMA,
    ],
)
def _sc_partials(out_hbm, ind_hbm, mask_hbm, tgt_hbm,
                 psum_hbm, pcnt_hbm,
                 ind_v, mask_v, idx_v, pred_v, tgt_v,
                 psum_v, pcnt_v, sem_t, sem_i, sem_a, sem_b):
    wid = lax.axis_index("s") * NC + lax.axis_index("c")

    cp_t = pltpu.async_copy(tgt_hbm.at[pl.ds(wid * EPW, EPW)], tgt_v, sem_t)
    cp_i = pltpu.async_copy(ind_hbm.at[pl.ds(wid * K, K)], ind_v, sem_i)
    cp_m = pltpu.async_copy(mask_hbm.at[pl.ds(wid * K, K)], mask_v, sem_i)
    cp_i.wait()
    cp_m.wait()

    lanes = lax.iota(jnp.int32, L)
    base = wid * (C * HW)
    wvec = jnp.where(lanes % 2 == 0,
                     jnp.full((L,), 1.0, jnp.float32),
                     jnp.full((L,), WEIGHT_ANGLE, jnp.float32))
    sems = (sem_a, sem_b)

    vks = [ind_v[pl.ds(g * L, L)] + base for g in range(K // L)]
    mfs = [mask_v[pl.ds(g * L, L)].astype(jnp.float32) for g in range(K // L)]
    coefs = [mf * wvec for mf in mfs]
    cnt = mfs[0]
    for mf in mfs[1:]:
        cnt = cnt + mf

    def build(j):
        for cl in range(BLK // K):
            c = j * (BLK // K) + cl
            for g in range(K // L):
                idx_v[pl.ds(c * K + g * L, L)] = vks[g] + c * HW
        return pltpu.async_copy(out_hbm.at[idx_v.at[pl.ds(j * BLK, BLK)]],
                                pred_v.at[pl.ds(j * BLK, BLK)], sems[j % 2])

    def compute(j, acc):
        for cl in range(BLK // K):
            c = j * (BLK // K) + cl
            for g in range(K // L):
                off = c * K + g * L
                d = pred_v[pl.ds(off, L)] - tgt_v[pl.ds(off, L)]
                acc = acc + coefs[g] * jnp.abs(d)
        return acc

    acc = jnp.zeros((L,), jnp.float32)
    cps = [build(0)]
    cp_t.wait()
    for j in range(1, CB):
        cps.append(build(j))
        cps[j - 1].wait()
        acc = compute(j - 1, acc)
    cps[CB - 1].wait()
    acc = compute(CB - 1, acc)

    psum_v[...] = acc
    pcnt_v[...] = cnt
    pltpu.sync_copy(psum_v, psum_hbm.at[wid])
    pltpu.sync_copy(pcnt_v, pcnt_hbm.at[wid])


def _finish_body(ps_ref, pc_ref, o_ref):
    total = jnp.sum(ps_ref[...])
    count = jnp.sum(pc_ref[...])
    o_ref[...] = jnp.broadcast_to(total / (count * float(C) + 1e-4), (1, 1))


_finish = pl.pallas_call(
    _finish_body,
    out_shape=jax.ShapeDtypeStruct((1, 1), jnp.float32),
)


def kernel(output, mask, ind, target, freq_mask):
    del freq_mask
    psum, pcnt = _sc_partials(
        output.reshape(-1),
        ind.reshape(-1).astype(jnp.int32),
        mask.reshape(-1).astype(jnp.int32),
        target.transpose(0, 2, 1).reshape(-1),
    )
    return _finish(psum, pcnt)[0, 0]
